# baseline (speedup 1.0000x reference)
import functools

import jax
import jax.numpy as jnp
from jax import lax
from jax.experimental import pallas as pl
from jax.experimental.pallas import tpu as pltpu

N = 10000
IN = 128
HID = 32
OUT = 16
H1 = 8


def _decoder_body(z_row, z_col, out_ref):
    a = z_row[...]
    b = z_col[...]
    acc = lax.dot_general(a, b, (((1,), (1,)), ((), ())),
                          preferred_element_type=jnp.float32)
    out_ref[...] = jax.nn.sigmoid(acc)


def _decoder(zp):
    BM = 512
    BN = 512
    grid = (pl.cdiv(N, BM), pl.cdiv(N, BN))
    return pl.pallas_call(
        _decoder_body,
        grid=grid,
        in_specs=[
            pl.BlockSpec((BM, 128), lambda i, j: (i, 0)),
            pl.BlockSpec((BN, 128), lambda i, j: (j, 0)),
        ],
        out_specs=pl.BlockSpec((BM, BN), lambda i, j: (i, j)),
        out_shape=jax.ShapeDtypeStruct((N, N), jnp.float32),
    )(zp, zp)


def _gatv2_layer(x, src, dst, Wl, bl, Wr, br, att, bias, heads, out_ch, concat,
                 num_nodes):
    xl = (x @ Wl + bl).reshape(num_nodes, heads, out_ch)
    xr = (x @ Wr + br).reshape(num_nodes, heads, out_ch)
    xj = jnp.take(xl, src, axis=0)
    xi = jnp.take(xr, dst, axis=0)
    e = jax.nn.leaky_relu(xi + xj, negative_slope=0.2)
    logits = (e * att[None, :, :]).sum(-1)
    m = jax.ops.segment_max(logits, dst, num_segments=num_nodes)
    expv = jnp.exp(logits - jnp.take(m, dst, axis=0))
    s = jax.ops.segment_sum(expv, dst, num_segments=num_nodes)
    alpha = expv / (jnp.take(s, dst, axis=0) + 1e-16)
    out = jax.ops.segment_sum(alpha[:, :, None] * xj, dst, num_segments=num_nodes)
    if concat:
        out = out.reshape(num_nodes, heads * out_ch)
    else:
        out = out.mean(axis=1)
    return out + bias


def kernel(x, edge_index, W1l, b1l, W1r, b1r, att1, bias1,
           W2l, b2l, W2r, b2r, att2, bias2):
    num_nodes = x.shape[0]
    loop = jnp.arange(num_nodes, dtype=edge_index.dtype)
    src = jnp.concatenate([edge_index[0], loop])
    dst = jnp.concatenate([edge_index[1], loop])
    h = _gatv2_layer(x, src, dst, W1l, b1l, W1r, b1r, att1, bias1, H1, HID,
                     True, num_nodes)
    h = _gatv2_layer(h, src, dst, W2l, b2l, W2r, b2r, att2, bias2, 1, OUT,
                     False, num_nodes)
    z = h / jnp.maximum(jnp.linalg.norm(h, axis=1, keepdims=True), 1e-12)
    zp = jnp.pad(z, ((0, 0), (0, 128 - OUT)))
    A_pred = _decoder(zp)
    return (A_pred, z)


# speedup vs baseline: 9.7880x; 9.7880x over previous
import functools

import jax
import jax.numpy as jnp
from jax import lax
from jax.experimental import pallas as pl
from jax.experimental.pallas import tpu as pltpu
from jax.experimental.pallas import tpu_sc as plsc

N = 10000
IN = 128
HID = 32
OUT = 16
H1 = 8
E = 320000

_L = 16
_G = 80
_NSUB = 16


def _iota16():
    return lax.broadcasted_iota(jnp.int32, (_L,), 0)


def _edge_chunk_compute(A, B, D, wbuf, attv, n_heads, chw):
    lane = _iota16()
    lanemod = lane % n_heads
    headmask = (lane < n_heads).astype(jnp.float32)
    n_cb = (n_heads * chw) // _L

    def group(g, _):
        rows = g * _L + lane

        for h in range(n_heads):
            def cbody(c, acc):
                colv = jnp.full((_L,), h * chw + c, jnp.int32)
                a = plsc.load_gather(A, [rows, colv])
                b = plsc.load_gather(B, [rows, colv])
                t = a + b
                t = jnp.where(t >= 0.0, t, t * 0.2)
                av = plsc.load_gather(attv, [colv])
                return acc + av * t

            acc = lax.fori_loop(0, chw, cbody, jnp.zeros((_L,), jnp.float32))
            wbuf[h, :] = jnp.exp(acc)

        def ebody(e, _):
            eab = g * _L + e
            ev = jnp.full((_L,), e, jnp.int32)
            for cb in range(n_cb):
                h = (cb * _L) // chw
                wv = plsc.load_gather(
                    wbuf, [jnp.full((_L,), h, jnp.int32), ev])
                sl = pl.ds(cb * _L, _L)
                A[eab, sl] = A[eab, sl] * wv
            wrow = plsc.load_gather(wbuf, [lanemod, ev])
            D[eab, :] = wrow * headmask
            return 0

        lax.fori_loop(0, _L, ebody, 0)
        return 0

    lax.fori_loop(0, _G // _L, group, 0)


def _sc_core_run(xl, xr, n0, d0, nout, dout, att, src_h, dst_h,
                 numer_sh, denom_sh, A, B, D, srcb, dstb, wbuf, attv, sem,
                 sid, edge_base, edges_per_tile, n_heads, chw):
    pltpu.sync_copy(att, attv)

    @pl.when(sid == 0)
    def _():
        pltpu.sync_copy(n0, numer_sh)
        pltpu.sync_copy(d0, denom_sh)

    plsc.subcore_barrier()

    base0 = edge_base + sid * edges_per_tile

    def chunk(i, _):
        base = base0 + i * _G
        pltpu.sync_copy(src_h.at[pl.ds(base, _G)], srcb)
        pltpu.sync_copy(dst_h.at[pl.ds(base, _G)], dstb)
        pltpu.async_copy(xl.at[srcb], A, sem).wait()
        pltpu.async_copy(xr.at[dstb], B, sem).wait()
        _edge_chunk_compute(A, B, D, wbuf, attv, n_heads, chw)
        pltpu.sync_copy(A, numer_sh.at[dstb], add=True)
        pltpu.sync_copy(D, denom_sh.at[dstb], add=True)
        return 0

    lax.fori_loop(0, edges_per_tile // _G, chunk, 0)
    plsc.subcore_barrier()

    @pl.when(sid == 0)
    def _():
        pltpu.sync_copy(numer_sh, nout)
        pltpu.sync_copy(denom_sh, dout)


def _l1_body(xl0, xl1, xr0, xr1, n00, n01, d00, d01, src_h, dst_h, att0, att1,
             nout0, nout1, dout0, dout1,
             numer_sh, denom_sh, A, B, D, srcb, dstb, wbuf, attv, sem):
    cid = lax.axis_index("c")
    sid = lax.axis_index("s")
    ept = E // _NSUB

    @pl.when(cid == 0)
    def _():
        _sc_core_run(xl0, xr0, n00, d00, nout0, dout0, att0, src_h, dst_h,
                     numer_sh, denom_sh, A, B, D, srcb, dstb, wbuf, attv, sem,
                     sid, 0, ept, 4, HID)

    @pl.when(cid == 1)
    def _():
        _sc_core_run(xl1, xr1, n01, d01, nout1, dout1, att1, src_h, dst_h,
                     numer_sh, denom_sh, A, B, D, srcb, dstb, wbuf, attv, sem,
                     sid, 0, ept, 4, HID)


def _l2_body(xl, xr, n0a, n0b, d0a, d0b, src_h, dst_h, att,
             nout0, nout1, dout0, dout1,
             numer_sh, denom_sh, A, B, D, srcb, dstb, wbuf, attv, sem):
    cid = lax.axis_index("c")
    sid = lax.axis_index("s")
    ept = (E // 2) // _NSUB

    @pl.when(cid == 0)
    def _():
        _sc_core_run(xl, xr, n0a, d0a, nout0, dout0, att, src_h, dst_h,
                     numer_sh, denom_sh, A, B, D, srcb, dstb, wbuf, attv, sem,
                     sid, 0, ept, 1, OUT)

    @pl.when(cid == 1)
    def _():
        _sc_core_run(xl, xr, n0b, d0b, nout1, dout1, att, src_h, dst_h,
                     numer_sh, denom_sh, A, B, D, srcb, dstb, wbuf, attv, sem,
                     sid, E // 2, ept, 1, OUT)


def _make_edge_pass(body, ch):
    f32 = jnp.float32
    mesh = plsc.VectorSubcoreMesh(core_axis_name="c", subcore_axis_name="s")
    return pl.kernel(
        body,
        out_type=(
            jax.ShapeDtypeStruct((N, ch), f32),
            jax.ShapeDtypeStruct((N, ch), f32),
            jax.ShapeDtypeStruct((N, _L), f32),
            jax.ShapeDtypeStruct((N, _L), f32),
        ),
        mesh=mesh,
        scratch_types=[
            pltpu.VMEM_SHARED((N, ch), f32),
            pltpu.VMEM_SHARED((N, _L), f32),
            pltpu.VMEM((_G, ch), f32),
            pltpu.VMEM((_G, ch), f32),
            pltpu.VMEM((_G, _L), f32),
            pltpu.VMEM((_G,), jnp.int32),
            pltpu.VMEM((_G,), jnp.int32),
            pltpu.VMEM((8, _L), f32),
            pltpu.VMEM((ch,), f32),
            pltpu.SemaphoreType.DMA,
        ],
        compiler_params=pltpu.CompilerParams(use_tc_tiling_on_sc=False,
                                             needs_layout_passes=False),
    )


_l1_pass = _make_edge_pass(_l1_body, 128)
_l2_pass = _make_edge_pass(_l2_body, OUT)


def _decoder_body(z_row, z_col, out_ref):
    acc = lax.dot_general(z_row[...], z_col[...], (((1,), (1,)), ((), ())),
                          preferred_element_type=jnp.float32)
    out_ref[...] = jax.nn.sigmoid(acc)


def _decoder(zp):
    BM = 512
    BN = 512
    grid = (pl.cdiv(N, BM), pl.cdiv(N, BN))
    return pl.pallas_call(
        _decoder_body,
        grid=grid,
        in_specs=[
            pl.BlockSpec((BM, 128), lambda i, j: (i, 0)),
            pl.BlockSpec((BN, 128), lambda i, j: (j, 0)),
        ],
        out_specs=pl.BlockSpec((BM, BN), lambda i, j: (i, j)),
        out_shape=jax.ShapeDtypeStruct((N, N), jnp.float32),
    )(zp, zp)


def _self_loop_init(xl, xr, att, n_heads, chw):
    xlh = xl.reshape(N, n_heads, chw)
    xrh = xr.reshape(N, n_heads, chw)
    t = xlh + xrh
    t = jnp.where(t >= 0.0, t, t * 0.2)
    sw = jnp.exp((t * att.reshape(1, n_heads, chw)).sum(-1))
    numer0 = (sw[:, :, None] * xlh).reshape(N, n_heads * chw)
    lane = jnp.arange(_L)
    denom0 = jnp.where(lane[None, :] < n_heads,
                       sw[:, lane % n_heads], 0.0).astype(jnp.float32)
    return numer0, denom0


def kernel(x, edge_index, W1l, b1l, W1r, b1r, att1, bias1,
           W2l, b2l, W2r, b2r, att2, bias2):
    src = edge_index[0]
    dst = edge_index[1]

    xl = x @ W1l + b1l
    xr = x @ W1r + b1r
    att1f = att1.reshape(H1 * HID)

    n00, d00 = _self_loop_init(xl[:, :128], xr[:, :128], att1f[:128], 4, HID)
    n01, d01 = _self_loop_init(xl[:, 128:], xr[:, 128:], att1f[128:], 4, HID)

    nout0, nout1, dout0, dout1 = _l1_pass(
        xl[:, :128], xl[:, 128:], xr[:, :128], xr[:, 128:],
        n00, n01, d00, d01, src, dst, att1f[:128], att1f[128:])

    numer = jnp.concatenate([nout0, nout1], axis=1).reshape(N, H1, HID)
    denom = jnp.concatenate([dout0[:, :4], dout1[:, :4]], axis=1)
    h1 = (numer / (denom[:, :, None] + 1e-16)).reshape(N, H1 * HID) + bias1

    xl2 = h1 @ W2l + b2l
    xr2 = h1 @ W2r + b2r
    att2f = att2.reshape(OUT)

    n0a, d0a = _self_loop_init(xl2, xr2, att2f, 1, OUT)
    zN16 = jnp.zeros((N, OUT), jnp.float32)
    zN = jnp.zeros((N, _L), jnp.float32)

    n20, n21, d20, d21 = _l2_pass(
        xl2, xr2, n0a, zN16, d0a, zN, src, dst, att2f)

    h2 = (n20 + n21) / (d20[:, :1] + d21[:, :1] + 1e-16) + bias2

    z = h2 / jnp.maximum(jnp.linalg.norm(h2, axis=1, keepdims=True), 1e-12)
    zp = jnp.pad(z, ((0, 0), (0, 128 - OUT)))
    A_pred = _decoder(zp)
    return (A_pred, z)


# speedup vs baseline: 14.8270x; 1.5148x over previous
import functools

import jax
import jax.numpy as jnp
from jax import lax
from jax.experimental import pallas as pl
from jax.experimental.pallas import tpu as pltpu
from jax.experimental.pallas import tpu_sc as plsc

N = 10000
IN = 128
HID = 32
OUT = 16
H1 = 8
E = 320000

_L = 16
_G = 80
_NSUB = 16


def _iota16():
    return lax.broadcasted_iota(jnp.int32, (_L,), 0)


def _group_compute(A, B, D, wbuf, attv, g, n_heads, chw):
    lane = _iota16()
    lanemod = lane % n_heads
    headmask = (lane < n_heads).astype(jnp.float32)
    rows = g * _L + lane
    n_cb = (n_heads * chw) // _L

    ws = []
    for h in range(n_heads):
        avs = [attv[pl.ds(h * chw + k * _L, _L)] for k in range(chw // _L)]
        acc = jnp.zeros((_L,), jnp.float32)
        for c in range(chw):
            ci = h * chw + c
            colv = jnp.full((_L,), ci, jnp.int32)
            a = plsc.load_gather(A, [rows, colv])
            b = plsc.load_gather(B, [rows, colv])
            t = a + b
            t = jnp.where(t >= 0.0, t, t * 0.2)
            acc = acc + avs[c // _L][c % _L] * t
        w = jnp.exp(acc)
        wbuf[h, :] = w
        ws.append(w)

    for e in range(_L):
        eab = g * _L + e
        ev = jnp.full((_L,), e, jnp.int32)
        for cb in range(n_cb):
            h = (cb * _L) // chw
            sl = pl.ds(cb * _L, _L)
            A[eab, sl] = A[eab, sl] * ws[h][e]
        wrow = plsc.load_gather(wbuf, [lanemod, ev])
        D[eab, :] = wrow * headmask


def _sc_core_run(xl, xr, n0, d0, nout, dout, att, src_h, dst_h,
                 numer_sh, denom_sh, A3, B3, D3, dsml3, srcstg, dststg,
                 wbuf, attv, gsa, gsb, ssn, ssd,
                 sid, edge_base, ept, n_heads, chw, G, stg):
    nchunks = ept // G
    base0 = edge_base + sid * ept

    pltpu.sync_copy(att, attv)

    @pl.when(sid == 0)
    def _():
        pltpu.sync_copy(n0, numer_sh)
        pltpu.sync_copy(d0, denom_sh)

    def stage(si):
        pltpu.sync_copy(src_h.at[pl.ds(base0 + si * stg * G, stg * G)],
                        srcstg)
        pltpu.sync_copy(dst_h.at[pl.ds(base0 + si * stg * G, stg * G)],
                        dststg)

    def fill_dsml(i, b):
        off = (i % stg) * G
        for j in range(G // _L):
            dsml3[b][pl.ds(j * _L, _L)] = dststg[pl.ds(off + j * _L, _L)]

    def gather_descs(i, b):
        off = (i % stg) * G
        return (
            pltpu.make_async_copy(xl.at[srcstg.at[pl.ds(off, G)]],
                                  A3[b], gsa[b]),
            pltpu.make_async_copy(xr.at[dststg.at[pl.ds(off, G)]],
                                  B3[b], gsb[b]),
        )

    def issue(i, b):
        fill_dsml(i, b)
        da, db = gather_descs(i, b)
        da.start()
        db.start()

    def wait_gathers(i, b):
        da, db = gather_descs(i, b)
        da.wait()
        db.wait()

    def issue_scatter(b):
        pltpu.async_copy(A3[b], numer_sh.at[dsml3[b]], ssn[b], add=True)
        pltpu.async_copy(D3[b], denom_sh.at[dsml3[b]], ssd[b], add=True)

    def wait_scatter(b):
        pltpu.make_async_copy(A3[b], numer_sh.at[dsml3[b]], ssn[b]).wait()
        pltpu.make_async_copy(D3[b], denom_sh.at[dsml3[b]], ssd[b]).wait()

    plsc.subcore_barrier()
    stage(0)
    issue(0, 0)

    def step(i, b):
        nb = (b + 1) % 3
        wait_gathers(i, b)

        @pl.when(i + 1 < nchunks)
        def _():
            @pl.when(i >= 2)
            def _():
                wait_scatter(nb)

            @pl.when((i + 1) % stg == 0)
            def _():
                stage((i + 1) // stg)

            issue(i + 1, nb)

        def group(g, _):
            _group_compute(A3[b], B3[b], D3[b], wbuf, attv, g, n_heads, chw)
            return 0

        lax.fori_loop(0, G // _L, group, 0)
        issue_scatter(b)

    def triple(p, _):
        step(3 * p, 0)
        step(3 * p + 1, 1)
        step(3 * p + 2, 2)
        return 0

    lax.fori_loop(0, nchunks // 3, triple, 0)
    for k in range(nchunks % 3):
        step(3 * (nchunks // 3) + k, k)

    for k in range(3):
        wait_scatter((nchunks - 3 + k) % 3)
    plsc.subcore_barrier()

    @pl.when(sid == 0)
    def _():
        pltpu.sync_copy(numer_sh, nout)
        pltpu.sync_copy(denom_sh, dout)


_G1 = 32
_G2 = 80
_STG = 25


def _l1_body(xl0, xl1, xr0, xr1, n00, n01, d00, d01, src_h, dst_h, att0, att1,
             nout0, nout1, dout0, dout1,
             numer_sh, denom_sh, A3, B3, D3, dsml3, srcstg, dststg,
             wbuf, attv, gsa, gsb, ssn, ssd):
    cid = lax.axis_index("c")
    sid = lax.axis_index("s")
    ept = E // _NSUB

    @pl.when(cid == 0)
    def _():
        _sc_core_run(xl0, xr0, n00, d00, nout0, dout0, att0, src_h, dst_h,
                     numer_sh, denom_sh, A3, B3, D3, dsml3, srcstg, dststg,
                     wbuf, attv, gsa, gsb, ssn, ssd,
                     sid, 0, ept, 4, HID, _G1, _STG)

    @pl.when(cid == 1)
    def _():
        _sc_core_run(xl1, xr1, n01, d01, nout1, dout1, att1, src_h, dst_h,
                     numer_sh, denom_sh, A3, B3, D3, dsml3, srcstg, dststg,
                     wbuf, attv, gsa, gsb, ssn, ssd,
                     sid, 0, ept, 4, HID, _G1, _STG)


def _l2_body(xl, xr, n0a, n0b, d0a, d0b, src_h, dst_h, att,
             nout0, nout1, dout0, dout1,
             numer_sh, denom_sh, A3, B3, D3, dsml3, srcstg, dststg,
             wbuf, attv, gsa, gsb, ssn, ssd):
    cid = lax.axis_index("c")
    sid = lax.axis_index("s")
    ept = (E // 2) // _NSUB

    @pl.when(cid == 0)
    def _():
        _sc_core_run(xl, xr, n0a, d0a, nout0, dout0, att, src_h, dst_h,
                     numer_sh, denom_sh, A3, B3, D3, dsml3, srcstg, dststg,
                     wbuf, attv, gsa, gsb, ssn, ssd,
                     sid, 0, ept, 1, OUT, _G2, _STG)

    @pl.when(cid == 1)
    def _():
        _sc_core_run(xl, xr, n0b, d0b, nout1, dout1, att, src_h, dst_h,
                     numer_sh, denom_sh, A3, B3, D3, dsml3, srcstg, dststg,
                     wbuf, attv, gsa, gsb, ssn, ssd,
                     sid, E // 2, ept, 1, OUT, _G2, _STG)


def _make_edge_pass(body, ch, G, stg):
    f32 = jnp.float32
    i32 = jnp.int32
    mesh = plsc.VectorSubcoreMesh(core_axis_name="c", subcore_axis_name="s")
    return pl.kernel(
        body,
        out_type=(
            jax.ShapeDtypeStruct((N, ch), f32),
            jax.ShapeDtypeStruct((N, ch), f32),
            jax.ShapeDtypeStruct((N, _L), f32),
            jax.ShapeDtypeStruct((N, _L), f32),
        ),
        mesh=mesh,
        scratch_types=[
            pltpu.VMEM_SHARED((N, ch), f32),
            pltpu.VMEM_SHARED((N, _L), f32),
            [pltpu.VMEM((G, ch), f32)] * 3,
            [pltpu.VMEM((G, ch), f32)] * 3,
            [pltpu.VMEM((G, _L), f32)] * 3,
            [pltpu.VMEM((G,), i32)] * 3,
            pltpu.VMEM((stg * G,), i32),
            pltpu.VMEM((stg * G,), i32),
            pltpu.VMEM((8, _L), f32),
            pltpu.VMEM((ch,), f32),
            [pltpu.SemaphoreType.DMA] * 3,
            [pltpu.SemaphoreType.DMA] * 3,
            [pltpu.SemaphoreType.DMA] * 3,
            [pltpu.SemaphoreType.DMA] * 3,
        ],
        compiler_params=pltpu.CompilerParams(use_tc_tiling_on_sc=False,
                                             needs_layout_passes=False),
    )


_l1_pass = _make_edge_pass(_l1_body, 128, _G1, _STG)
_l2_pass = _make_edge_pass(_l2_body, OUT, _G2, _STG)


def _decoder_body(z_row, z_col, out_ref):
    acc = lax.dot_general(z_row[...], z_col[...], (((1,), (1,)), ((), ())),
                          preferred_element_type=jnp.float32)
    out_ref[...] = jax.nn.sigmoid(acc)


def _decoder(zp):
    BM = 512
    BN = 512
    grid = (pl.cdiv(N, BM), pl.cdiv(N, BN))
    return pl.pallas_call(
        _decoder_body,
        grid=grid,
        in_specs=[
            pl.BlockSpec((BM, 128), lambda i, j: (i, 0)),
            pl.BlockSpec((BN, 128), lambda i, j: (j, 0)),
        ],
        out_specs=pl.BlockSpec((BM, BN), lambda i, j: (i, j)),
        out_shape=jax.ShapeDtypeStruct((N, N), jnp.float32),
    )(zp, zp)


def _self_loop_init(xl, xr, att, n_heads, chw):
    xlh = xl.reshape(N, n_heads, chw)
    xrh = xr.reshape(N, n_heads, chw)
    t = xlh + xrh
    t = jnp.where(t >= 0.0, t, t * 0.2)
    sw = jnp.exp((t * att.reshape(1, n_heads, chw)).sum(-1))
    numer0 = (sw[:, :, None] * xlh).reshape(N, n_heads * chw)
    lane = jnp.arange(_L)
    denom0 = jnp.where(lane[None, :] < n_heads,
                       sw[:, lane % n_heads], 0.0).astype(jnp.float32)
    return numer0, denom0


def kernel(x, edge_index, W1l, b1l, W1r, b1r, att1, bias1,
           W2l, b2l, W2r, b2r, att2, bias2):
    src = edge_index[0]
    dst = edge_index[1]

    xl = x @ W1l + b1l
    xr = x @ W1r + b1r
    att1f = att1.reshape(H1 * HID)

    n00, d00 = _self_loop_init(xl[:, :128], xr[:, :128], att1f[:128], 4, HID)
    n01, d01 = _self_loop_init(xl[:, 128:], xr[:, 128:], att1f[128:], 4, HID)

    nout0, nout1, dout0, dout1 = _l1_pass(
        xl[:, :128], xl[:, 128:], xr[:, :128], xr[:, 128:],
        n00, n01, d00, d01, src, dst, att1f[:128], att1f[128:])

    numer = jnp.concatenate([nout0, nout1], axis=1).reshape(N, H1, HID)
    denom = jnp.concatenate([dout0[:, :4], dout1[:, :4]], axis=1)
    h1 = (numer / (denom[:, :, None] + 1e-16)).reshape(N, H1 * HID) + bias1

    xl2 = h1 @ W2l + b2l
    xr2 = h1 @ W2r + b2r
    att2f = att2.reshape(OUT)

    n0a, d0a = _self_loop_init(xl2, xr2, att2f, 1, OUT)
    zN16 = jnp.zeros((N, OUT), jnp.float32)
    zN = jnp.zeros((N, _L), jnp.float32)

    n20, n21, d20, d21 = _l2_pass(
        xl2, xr2, n0a, zN16, d0a, zN, src, dst, att2f)

    h2 = (n20 + n21) / (d20[:, :1] + d21[:, :1] + 1e-16) + bias2

    z = h2 / jnp.maximum(jnp.linalg.norm(h2, axis=1, keepdims=True), 1e-12)
    zp = jnp.pad(z, ((0, 0), (0, 128 - OUT)))
    A_pred = _decoder(zp)
    return (A_pred, z)
